# baseline (speedup 1.0000x reference)
import jax
import jax.numpy as jnp
from jax import lax
from jax.experimental import pallas as pl
from jax.experimental.pallas import tpu as pltpu

B = 16
H = 16
D = 64
PAGE = 16
HD = H * D
MAXP = 128
G = 8
T = G * PAGE
C = MAXP // G
NW = B * C
NSLOT = 4
DEPTH = 3
NEG = -1e30


def _attn_body(wb_ref, wc_ref, tot_ref, bt_ref, cl_ref,
               qt_ref, key_ref, val_ref, pg_ref, rw_ref,
               kc_ref, vc_ref,
               out_ref,
               k_buf, v_buf, acc_ref, m_ref, l_ref,
               spat_ref, qbd_ref, pv_ref, sems):
    t = pl.program_id(0)
    total = tot_ref[0]

    def chunk_copies(tt):
        s = lax.rem(tt, NSLOT)
        bb = wb_ref[tt]
        cc = wc_ref[tt]
        cps = []
        for g in range(G):
            page = bt_ref[bb, cc * G + g]
            cps.append(pltpu.make_async_copy(
                kc_ref.at[page], k_buf.at[s, pl.ds(g * PAGE, PAGE), :],
                sems.at[s]))
            cps.append(pltpu.make_async_copy(
                vc_ref.at[page], v_buf.at[s, pl.ds(g * PAGE, PAGE), :],
                sems.at[s]))
        return cps

    @pl.when(t == 0)
    def _prologue():
        for i in range(DEPTH):
            for cp in chunk_copies(jnp.int32(i)):
                cp.start()

    @pl.when(t < total)
    def _step():
        b = wb_ref[t]
        c = wc_ref[t]
        len_b = jnp.maximum(cl_ref[b], 1)

        for cp in chunk_copies(t):
            cp.wait()

        @pl.when(t + DEPTH < total)
        def _issue_ahead():
            for cp in chunk_copies(t + DEPTH):
                cp.start()

        @pl.when(c == 0)
        def _init_seq():
            m_ref[...] = jnp.full_like(m_ref, NEG)
            l_ref[...] = jnp.zeros_like(l_ref)
            acc_ref[...] = jnp.zeros_like(acc_ref)
            hsel = (lax.broadcasted_iota(jnp.int32, (HD, H), 0) // D
                    == lax.broadcasted_iota(jnp.int32, (HD, H), 1))
            qbd_ref[...] = jnp.where(hsel, qt_ref[0], 0.0).astype(jnp.bfloat16)
            spat_ref[...] = jnp.dot(key_ref[...], qbd_ref[...],
                                    preferred_element_type=jnp.float32)

        s = lax.rem(t, NSLOT)
        for g in range(G):
            pv_ref[pl.ds(g * PAGE, PAGE), :] = jnp.full(
                (PAGE, 1), bt_ref[b, c * G + g], dtype=jnp.int32)

        k_chunk = k_buf[s]
        v_chunk = v_buf[s]

        rmod = lax.broadcasted_iota(jnp.int32, (T, 1), 0) % PAGE
        onehot_b = jnp.logical_and(pv_ref[...] == pg_ref[...],
                                   rmod == rw_ref[...])
        onehot = onehot_b.astype(jnp.float32)
        anyp = jnp.any(onehot_b, axis=1, keepdims=True)

        scores = jnp.dot(k_chunk, qbd_ref[...],
                         preferred_element_type=jnp.float32)
        s_pat = jnp.dot(onehot, spat_ref[...],
                        preferred_element_type=jnp.float32)
        scores = jnp.where(anyp, s_pat, scores)
        pos = c * T + lax.broadcasted_iota(jnp.int32, (T, 1), 0)
        scores = jnp.where(pos < len_b, scores, NEG)

        m_old = m_ref[...]
        m_new = jnp.maximum(m_old, jnp.max(scores, axis=0, keepdims=True))
        alpha = jnp.exp(m_old - m_new)
        p = jnp.exp(scores - m_new)
        l_ref[...] = l_ref[...] * alpha + jnp.sum(p, axis=0, keepdims=True)
        m_ref[...] = m_new

        p_pat = jnp.where(anyp, p, 0.0)
        p_unp = (p - p_pat).astype(jnp.bfloat16)
        dn = (((0,), (0,)), ((), ()))
        r = lax.dot_general(p_unp, v_chunk, dn,
                            preferred_element_type=jnp.float32)
        mix = lax.dot_general(p_pat, onehot, dn,
                              preferred_element_type=jnp.float32)
        r = r + jnp.dot(mix, val_ref[...], preferred_element_type=jnp.float32)

        e2 = (lax.broadcasted_iota(jnp.int32, (H, HD), 1) // D
              == lax.broadcasted_iota(jnp.int32, (H, HD), 0))
        e2f = e2.astype(jnp.float32)
        contrib = jnp.sum(jnp.where(e2, r, 0.0), axis=0, keepdims=True)
        alpha_e = jnp.dot(alpha, e2f, preferred_element_type=jnp.float32)
        acc_ref[...] = acc_ref[...] * alpha_e + contrib

        @pl.when(wc_ref[t + 1] == 0)
        def _finalize():
            l_e = jnp.dot(l_ref[...], e2f, preferred_element_type=jnp.float32)
            out_ref[0] = acc_ref[...] / l_e


def kernel(query, key, value, key_cache, value_cache, slot_mapping,
           block_tables, context_lens):
    scale = 1.0 / jnp.sqrt(jnp.asarray(D, dtype=jnp.float32))
    kc = key_cache.reshape(key_cache.shape[0], PAGE, HD).astype(jnp.bfloat16)
    vc = value_cache.reshape(value_cache.shape[0], PAGE, HD).astype(jnp.bfloat16)
    qt = query.reshape(B, HD, 1) * scale
    key2 = key.reshape(B, HD)
    val2 = value.reshape(B, HD)
    sm = slot_mapping.astype(jnp.int32)
    jidx = jnp.arange(B, dtype=jnp.int32)
    has_later = jnp.any((sm[None, :] == sm[:, None])
                        & (jidx[None, :] > jidx[:, None]), axis=1)
    pg = jnp.where(has_later, -1, sm // PAGE).reshape(1, B)
    rw = (sm % PAGE).reshape(1, B)

    cl = context_lens.astype(jnp.int32)
    n_chunks = (jnp.maximum(cl, 1) + (T - 1)) // T
    starts = jnp.cumsum(n_chunks) - n_chunks
    total = jnp.sum(n_chunks).reshape(1)
    tidx = jnp.arange(NW, dtype=jnp.int32)
    wb = jnp.sum((tidx[:, None] >= (starts + n_chunks)[None, :]).astype(
        jnp.int32), axis=1)
    wb = jnp.minimum(wb, B - 1)
    wc = tidx - starts[wb]
    wc = jnp.where(tidx < total[0], wc, 0)
    wb = jnp.concatenate([wb, jnp.array([B - 1], jnp.int32)])
    wc = jnp.concatenate([wc, jnp.array([0], jnp.int32)])

    grid_spec = pltpu.PrefetchScalarGridSpec(
        num_scalar_prefetch=5,
        grid=(NW,),
        in_specs=[
            pl.BlockSpec((1, HD, 1), lambda t, *s: (s[0][t], 0, 0)),
            pl.BlockSpec((B, HD), lambda t, *s: (0, 0)),
            pl.BlockSpec((B, HD), lambda t, *s: (0, 0)),
            pl.BlockSpec((1, B), lambda t, *s: (0, 0)),
            pl.BlockSpec((1, B), lambda t, *s: (0, 0)),
            pl.BlockSpec(memory_space=pl.ANY),
            pl.BlockSpec(memory_space=pl.ANY),
        ],
        out_specs=pl.BlockSpec((1, 1, HD), lambda t, *s: (s[0][t], 0, 0)),
        scratch_shapes=[
            pltpu.VMEM((NSLOT, T, HD), jnp.bfloat16),
            pltpu.VMEM((NSLOT, T, HD), jnp.bfloat16),
            pltpu.VMEM((1, HD), jnp.float32),
            pltpu.VMEM((1, H), jnp.float32),
            pltpu.VMEM((1, H), jnp.float32),
            pltpu.VMEM((B, H), jnp.float32),
            pltpu.VMEM((HD, H), jnp.bfloat16),
            pltpu.VMEM((T, 1), jnp.int32),
            pltpu.SemaphoreType.DMA((NSLOT,)),
        ],
    )
    out = pl.pallas_call(
        _attn_body,
        grid_spec=grid_spec,
        out_shape=jax.ShapeDtypeStruct((B, 1, HD), jnp.float32),
        compiler_params=pltpu.CompilerParams(
            dimension_semantics=("arbitrary",),
        ),
    )(wb, wc, total, block_tables, cl, qt, key2, val2, pg, rw, kc, vc)
    return out.reshape(B, H, D)


# speedup vs baseline: 1.0481x; 1.0481x over previous
import jax
import jax.numpy as jnp
from jax import lax
from jax.experimental import pallas as pl
from jax.experimental.pallas import tpu as pltpu

B = 16
H = 16
D = 64
PAGE = 16
HD = H * D
MAXP = 128
G = 8
T = G * PAGE
C = MAXP // G
NW = B * C
NSLOT = 4
DEPTH = 3
NEG = -1e30


def _attn_body(wb_ref, wc_ref, tot_ref, bt_ref, cl_ref,
               qt_ref, key_ref, val_ref, pg_ref, rw_ref,
               kvc_ref,
               out_ref,
               kv_buf, acc_ref, m_ref, l_ref,
               spat_ref, qbd_ref, pv_ref, sems):
    t = pl.program_id(0)
    total = tot_ref[0]

    def chunk_copies(tt):
        s = lax.rem(tt, NSLOT)
        bb = wb_ref[tt]
        cc = wc_ref[tt]
        cps = []
        for g in range(G):
            page = bt_ref[bb, cc * G + g]
            cps.append(pltpu.make_async_copy(
                kvc_ref.at[page], kv_buf.at[s, :, pl.ds(g * PAGE, PAGE), :],
                sems.at[s]))
        return cps

    @pl.when(t == 0)
    def _prologue():
        for i in range(DEPTH):
            for cp in chunk_copies(jnp.int32(i)):
                cp.start()

    @pl.when(t < total)
    def _step():
        b = wb_ref[t]
        c = wc_ref[t]
        len_b = jnp.maximum(cl_ref[b], 1)

        for cp in chunk_copies(t):
            cp.wait()

        @pl.when(t + DEPTH < total)
        def _issue_ahead():
            for cp in chunk_copies(t + DEPTH):
                cp.start()

        @pl.when(c == 0)
        def _init_seq():
            m_ref[...] = jnp.full_like(m_ref, NEG)
            l_ref[...] = jnp.zeros_like(l_ref)
            acc_ref[...] = jnp.zeros_like(acc_ref)
            hsel = (lax.broadcasted_iota(jnp.int32, (HD, H), 0) // D
                    == lax.broadcasted_iota(jnp.int32, (HD, H), 1))
            qbd_ref[...] = jnp.where(hsel, qt_ref[0], 0.0).astype(jnp.bfloat16)
            spat_ref[...] = jnp.dot(key_ref[...], qbd_ref[...],
                                    preferred_element_type=jnp.float32)

        s = lax.rem(t, NSLOT)
        for g in range(G):
            pv_ref[pl.ds(g * PAGE, PAGE), :] = jnp.full(
                (PAGE, 1), bt_ref[b, c * G + g], dtype=jnp.int32)

        k_chunk = kv_buf[s, 0]
        v_chunk = kv_buf[s, 1]

        rmod = lax.broadcasted_iota(jnp.int32, (T, 1), 0) % PAGE
        onehot_b = jnp.logical_and(pv_ref[...] == pg_ref[...],
                                   rmod == rw_ref[...])
        onehot = onehot_b.astype(jnp.float32)
        anyp = jnp.any(onehot_b, axis=1, keepdims=True)

        scores = jnp.dot(k_chunk, qbd_ref[...],
                         preferred_element_type=jnp.float32)
        s_pat = jnp.dot(onehot, spat_ref[...],
                        preferred_element_type=jnp.float32)
        scores = jnp.where(anyp, s_pat, scores)
        pos = c * T + lax.broadcasted_iota(jnp.int32, (T, 1), 0)
        scores = jnp.where(pos < len_b, scores, NEG)

        m_old = m_ref[...]
        m_new = jnp.maximum(m_old, jnp.max(scores, axis=0, keepdims=True))
        alpha = jnp.exp(m_old - m_new)
        p = jnp.exp(scores - m_new)
        l_ref[...] = l_ref[...] * alpha + jnp.sum(p, axis=0, keepdims=True)
        m_ref[...] = m_new

        p_pat = jnp.where(anyp, p, 0.0)
        p_unp = (p - p_pat).astype(jnp.bfloat16)
        dn = (((0,), (0,)), ((), ()))
        r = lax.dot_general(p_unp, v_chunk, dn,
                            preferred_element_type=jnp.float32)
        mix = lax.dot_general(p_pat, onehot, dn,
                              preferred_element_type=jnp.float32)
        r = r + jnp.dot(mix, val_ref[...], preferred_element_type=jnp.float32)

        e2 = (lax.broadcasted_iota(jnp.int32, (H, HD), 1) // D
              == lax.broadcasted_iota(jnp.int32, (H, HD), 0))
        e2f = e2.astype(jnp.float32)
        contrib = jnp.sum(jnp.where(e2, r, 0.0), axis=0, keepdims=True)
        alpha_e = jnp.dot(alpha, e2f, preferred_element_type=jnp.float32)
        acc_ref[...] = acc_ref[...] * alpha_e + contrib

        @pl.when(wc_ref[t + 1] == 0)
        def _finalize():
            l_e = jnp.dot(l_ref[...], e2f, preferred_element_type=jnp.float32)
            out_ref[0] = acc_ref[...] / l_e


def kernel(query, key, value, key_cache, value_cache, slot_mapping,
           block_tables, context_lens):
    scale = 1.0 / jnp.sqrt(jnp.asarray(D, dtype=jnp.float32))
    kvc = jnp.stack(
        [key_cache.reshape(key_cache.shape[0], PAGE, HD),
         value_cache.reshape(value_cache.shape[0], PAGE, HD)],
        axis=1).astype(jnp.bfloat16)
    qt = query.reshape(B, HD, 1) * scale
    key2 = key.reshape(B, HD)
    val2 = value.reshape(B, HD)
    sm = slot_mapping.astype(jnp.int32)
    jidx = jnp.arange(B, dtype=jnp.int32)
    has_later = jnp.any((sm[None, :] == sm[:, None])
                        & (jidx[None, :] > jidx[:, None]), axis=1)
    pg = jnp.where(has_later, -1, sm // PAGE).reshape(1, B)
    rw = (sm % PAGE).reshape(1, B)

    cl = context_lens.astype(jnp.int32)
    n_chunks = (jnp.maximum(cl, 1) + (T - 1)) // T
    starts = jnp.cumsum(n_chunks) - n_chunks
    total = jnp.sum(n_chunks).reshape(1)
    tidx = jnp.arange(NW, dtype=jnp.int32)
    wb = jnp.sum((tidx[:, None] >= (starts + n_chunks)[None, :]).astype(
        jnp.int32), axis=1)
    wb = jnp.minimum(wb, B - 1)
    wc = tidx - starts[wb]
    wc = jnp.where(tidx < total[0], wc, 0)
    wb = jnp.concatenate([wb, jnp.array([B - 1], jnp.int32)])
    wc = jnp.concatenate([wc, jnp.array([0], jnp.int32)])

    grid_spec = pltpu.PrefetchScalarGridSpec(
        num_scalar_prefetch=5,
        grid=(NW,),
        in_specs=[
            pl.BlockSpec((1, HD, 1), lambda t, *s: (s[0][t], 0, 0)),
            pl.BlockSpec((B, HD), lambda t, *s: (0, 0)),
            pl.BlockSpec((B, HD), lambda t, *s: (0, 0)),
            pl.BlockSpec((1, B), lambda t, *s: (0, 0)),
            pl.BlockSpec((1, B), lambda t, *s: (0, 0)),
            pl.BlockSpec(memory_space=pl.ANY),
        ],
        out_specs=pl.BlockSpec((1, 1, HD), lambda t, *s: (s[0][t], 0, 0)),
        scratch_shapes=[
            pltpu.VMEM((NSLOT, 2, T, HD), jnp.bfloat16),
            pltpu.VMEM((1, HD), jnp.float32),
            pltpu.VMEM((1, H), jnp.float32),
            pltpu.VMEM((1, H), jnp.float32),
            pltpu.VMEM((B, H), jnp.float32),
            pltpu.VMEM((HD, H), jnp.bfloat16),
            pltpu.VMEM((T, 1), jnp.int32),
            pltpu.SemaphoreType.DMA((NSLOT,)),
        ],
    )
    out = pl.pallas_call(
        _attn_body,
        grid_spec=grid_spec,
        out_shape=jax.ShapeDtypeStruct((B, 1, HD), jnp.float32),
        compiler_params=pltpu.CompilerParams(
            dimension_semantics=("arbitrary",),
        ),
    )(wb, wc, total, block_tables, cl, qt, key2, val2, pg, rw, kvc)
    return out.reshape(B, H, D)


# speedup vs baseline: 1.1425x; 1.0900x over previous
import jax
import jax.numpy as jnp
from jax import lax
from jax.experimental import pallas as pl
from jax.experimental.pallas import tpu as pltpu

B = 16
H = 16
D = 64
PAGE = 16
HD = H * D
MAXP = 128
G = 16
T = G * PAGE
C = MAXP // G
NW = B * C
NSLOT = 4
DEPTH = 3
NEG = -1e30


def _attn_body(wb_ref, wc_ref, tot_ref, bt_ref, cl_ref, hp_ref,
               qbd_ref, key_ref, val_ref, pg_ref, rw_ref,
               kvc_ref,
               out_ref,
               kv_buf, acc_ref, m_ref, l_ref,
               spat_ref, sc_ref, r_ref, pv_ref, sems):
    t = pl.program_id(0)
    total = tot_ref[0]

    def chunk_copies(tt):
        s = lax.rem(tt, NSLOT)
        bb = wb_ref[tt]
        cc = wc_ref[tt]
        cps = []
        for g in range(G):
            page = bt_ref[bb, cc * G + g]
            cps.append(pltpu.make_async_copy(
                kvc_ref.at[page], kv_buf.at[s, :, pl.ds(g * PAGE, PAGE), :],
                sems.at[s]))
        return cps

    @pl.when(t == 0)
    def _prologue():
        for i in range(DEPTH):
            for cp in chunk_copies(jnp.int32(i)):
                cp.start()

    @pl.when(t < total)
    def _step():
        b = wb_ref[t]
        c = wc_ref[t]
        len_b = jnp.maximum(cl_ref[b], 1)

        for cp in chunk_copies(t):
            cp.wait()

        @pl.when(t + DEPTH < total)
        def _issue_ahead():
            for cp in chunk_copies(t + DEPTH):
                cp.start()

        @pl.when(c == 0)
        def _init_seq():
            m_ref[...] = jnp.full_like(m_ref, NEG)
            l_ref[...] = jnp.zeros_like(l_ref)
            acc_ref[...] = jnp.zeros_like(acc_ref)
            spat_ref[...] = jnp.dot(key_ref[...], qbd_ref[0],
                                    preferred_element_type=jnp.float32)

        s = lax.rem(t, NSLOT)
        k_chunk = kv_buf[s, 0]
        v_chunk = kv_buf[s, 1]

        sc_ref[...] = jnp.dot(k_chunk, qbd_ref[0],
                              preferred_element_type=jnp.float32)

        @pl.when(hp_ref[t] == 1)
        def _patch_scores():
            for g in range(G):
                pv_ref[pl.ds(g * PAGE, PAGE), :] = jnp.full(
                    (PAGE, 1), bt_ref[b, c * G + g], dtype=jnp.int32)
            rmod = lax.broadcasted_iota(jnp.int32, (T, 1), 0) % PAGE
            onehot_b = jnp.logical_and(pv_ref[...] == pg_ref[...],
                                       rmod == rw_ref[...])
            anyp = jnp.any(onehot_b, axis=1, keepdims=True)
            s_pat = jnp.dot(onehot_b.astype(jnp.float32), spat_ref[...],
                            preferred_element_type=jnp.float32)
            sc_ref[...] = jnp.where(anyp, s_pat, sc_ref[...])

        pos = c * T + lax.broadcasted_iota(jnp.int32, (T, 1), 0)
        scores = jnp.where(pos < len_b, sc_ref[...], NEG)

        m_old = m_ref[...]
        m_new = jnp.maximum(m_old, jnp.max(scores, axis=0, keepdims=True))
        alpha = jnp.exp(m_old - m_new)
        p = jnp.exp(scores - m_new)
        l_ref[...] = l_ref[...] * alpha + jnp.sum(p, axis=0, keepdims=True)
        m_ref[...] = m_new

        dn = (((0,), (0,)), ((), ()))
        r_ref[...] = lax.dot_general(p.astype(jnp.bfloat16), v_chunk, dn,
                                     preferred_element_type=jnp.float32)

        @pl.when(hp_ref[t] == 1)
        def _patch_pv():
            rmod = lax.broadcasted_iota(jnp.int32, (T, 1), 0) % PAGE
            onehot_b = jnp.logical_and(pv_ref[...] == pg_ref[...],
                                       rmod == rw_ref[...])
            anyp = jnp.any(onehot_b, axis=1, keepdims=True)
            onehot = onehot_b.astype(jnp.float32)
            p_pat = jnp.where(anyp, p, 0.0)
            mix = lax.dot_general(p_pat, onehot, dn,
                                  preferred_element_type=jnp.float32)
            corr = (jnp.dot(mix, val_ref[...],
                            preferred_element_type=jnp.float32)
                    - lax.dot_general(p_pat.astype(jnp.bfloat16), v_chunk, dn,
                                      preferred_element_type=jnp.float32))
            r_ref[...] = r_ref[...] + corr

        e2 = (lax.broadcasted_iota(jnp.int32, (H, HD), 1) // D
              == lax.broadcasted_iota(jnp.int32, (H, HD), 0))
        e2f = e2.astype(jnp.float32)
        contrib = jnp.sum(jnp.where(e2, r_ref[...], 0.0), axis=0,
                          keepdims=True)
        alpha_e = jnp.dot(alpha, e2f, preferred_element_type=jnp.float32)
        acc_ref[...] = acc_ref[...] * alpha_e + contrib

        @pl.when(wc_ref[t + 1] == 0)
        def _finalize():
            l_e = jnp.dot(l_ref[...], e2f, preferred_element_type=jnp.float32)
            out_ref[0] = acc_ref[...] / l_e


def kernel(query, key, value, key_cache, value_cache, slot_mapping,
           block_tables, context_lens):
    scale = 1.0 / jnp.sqrt(jnp.asarray(D, dtype=jnp.float32))
    kvc = jnp.stack(
        [key_cache.reshape(key_cache.shape[0], PAGE, HD),
         value_cache.reshape(value_cache.shape[0], PAGE, HD)],
        axis=1).astype(jnp.bfloat16)
    eye = jnp.eye(H, dtype=jnp.float32)
    qbd = ((query * scale)[:, :, :, None] * eye[:, None, :]
           ).reshape(B, HD, H).astype(jnp.bfloat16)
    key2 = key.reshape(B, HD).astype(jnp.bfloat16)
    val2 = value.reshape(B, HD)
    sm = slot_mapping.astype(jnp.int32)
    jidx = jnp.arange(B, dtype=jnp.int32)
    has_later = jnp.any((sm[None, :] == sm[:, None])
                        & (jidx[None, :] > jidx[:, None]), axis=1)
    pgs = jnp.where(has_later, -1, sm // PAGE)
    pg = pgs.reshape(1, B)
    rw = (sm % PAGE).reshape(1, B)

    cl = context_lens.astype(jnp.int32)
    n_chunks = (jnp.maximum(cl, 1) + (T - 1)) // T
    starts = jnp.cumsum(n_chunks) - n_chunks
    total = jnp.sum(n_chunks).reshape(1)
    tidx = jnp.arange(NW, dtype=jnp.int32)
    wb = jnp.sum((tidx[:, None] >= (starts + n_chunks)[None, :]).astype(
        jnp.int32), axis=1)
    wb = jnp.minimum(wb, B - 1)
    wc = tidx - starts[wb]
    wc = jnp.where(tidx < total[0], wc, 0)
    page_match = jnp.any(block_tables[:, :, None] == pgs[None, None, :],
                         axis=-1)
    chunk_has = jnp.any(page_match.reshape(B, C, G), axis=-1)
    hp = chunk_has[wb, wc].astype(jnp.int32)
    wb = jnp.concatenate([wb, jnp.array([B - 1], jnp.int32)])
    wc = jnp.concatenate([wc, jnp.array([0], jnp.int32)])

    grid_spec = pltpu.PrefetchScalarGridSpec(
        num_scalar_prefetch=6,
        grid=(NW,),
        in_specs=[
            pl.BlockSpec((1, HD, H), lambda t, *s: (s[0][t], 0, 0)),
            pl.BlockSpec((B, HD), lambda t, *s: (0, 0)),
            pl.BlockSpec((B, HD), lambda t, *s: (0, 0)),
            pl.BlockSpec((1, B), lambda t, *s: (0, 0)),
            pl.BlockSpec((1, B), lambda t, *s: (0, 0)),
            pl.BlockSpec(memory_space=pl.ANY),
        ],
        out_specs=pl.BlockSpec((1, 1, HD), lambda t, *s: (s[0][t], 0, 0)),
        scratch_shapes=[
            pltpu.VMEM((NSLOT, 2, T, HD), jnp.bfloat16),
            pltpu.VMEM((1, HD), jnp.float32),
            pltpu.VMEM((1, H), jnp.float32),
            pltpu.VMEM((1, H), jnp.float32),
            pltpu.VMEM((B, H), jnp.float32),
            pltpu.VMEM((T, H), jnp.float32),
            pltpu.VMEM((H, HD), jnp.float32),
            pltpu.VMEM((T, 1), jnp.int32),
            pltpu.SemaphoreType.DMA((NSLOT,)),
        ],
    )
    out = pl.pallas_call(
        _attn_body,
        grid_spec=grid_spec,
        out_shape=jax.ShapeDtypeStruct((B, 1, HD), jnp.float32),
        compiler_params=pltpu.CompilerParams(
            dimension_semantics=("arbitrary",),
        ),
    )(wb, wc, total, block_tables, cl, hp, qbd, key2, val2, pg, rw, kvc)
    return out.reshape(B, H, D)
